# Initial kernel scaffold; baseline (speedup 1.0000x reference)
import jax
import jax.numpy as jnp
from jax.experimental import pallas as pl


def kernel(node_feats, edge_feats, edge_index, W_node, b_node, W_edge, b_edge, W_msg, b_msg, W_upd, b_upd, W_lin1, b_lin1, W_lin2, b_lin2):
    raise NotImplementedError("write your pallas kernel here")


# speedup vs baseline: 3.0308x; 3.0308x over previous
import functools

import jax
import jax.numpy as jnp
from jax import lax
from jax.experimental import pallas as pl
from jax.experimental.pallas import tpu as pltpu
from jax.experimental.pallas import tpu_sc as plsc

F32 = jnp.float32

_N = 10000
_E = 320000
_ND = 128
_ED = 16
_H = 64
_L = 4
_T = 1

_NC = 2
_NS = 16
_NW = _NC * _NS
_EPW = _E // _NW
_K = 80
_NCHUNK = _EPW // _K
_RPT = _N // _NS


def _leaky(x):
    return jnp.maximum(x, 0.01 * x)


_BE = 8000


def _q_body(ef_ref, c_ref, d_ref, q_ref):
    x = ef_ref[...]
    for l in range(_L):
        q_ref[l] = jnp.dot(x, c_ref[l], preferred_element_type=F32) + d_ref[l]


def _q_call(edge_feats, C_all, d_all):
    return pl.pallas_call(
        _q_body,
        grid=(_E // _BE,),
        in_specs=[
            pl.BlockSpec((_BE, _ED), lambda i: (i, 0)),
            pl.BlockSpec((_L, _ED, _H), lambda i: (0, 0, 0)),
            pl.BlockSpec((_L, _H), lambda i: (0, 0)),
        ],
        out_specs=pl.BlockSpec((_L, _BE, _H), lambda i: (0, i, 0)),
        out_shape=jax.ShapeDtypeStruct((_L, _E, _H), F32),
    )(edge_feats, C_all, d_all)


def _embed_body(nf_ref, wn_ref, bn_ref, wm0_ref, h_ref, p_ref):
    h = jnp.dot(nf_ref[...], wn_ref[...], preferred_element_type=F32) + bn_ref[...]
    h_ref[...] = h
    p_ref[...] = jnp.dot(h, wm0_ref[...], preferred_element_type=F32)


def _embed_call(node_feats, W_node, b_node, W_msg0):
    return pl.pallas_call(
        _embed_body,
        out_shape=[
            jax.ShapeDtypeStruct((_N, _H), F32),
            jax.ShapeDtypeStruct((_N, _H), F32),
        ],
    )(node_feats, W_node, b_node.reshape(1, _H), W_msg0)


def _sc_body(p_hbm, q_hbm, s_hbm, d_hbm, z_hbm, out_hbm,
             p_tab, agg_tab, sidx, didx, qbuf, pbuf):
    c = lax.axis_index("c")
    s = lax.axis_index("s")
    row0 = s * _RPT
    pltpu.sync_copy(p_hbm.at[pl.ds(row0, _RPT)], p_tab.at[pl.ds(row0, _RPT)])
    pltpu.sync_copy(z_hbm.at[pl.ds(row0, _RPT)], agg_tab.at[pl.ds(row0, _RPT)])
    rowbase = (c * _NS + s) * _NCHUNK
    pltpu.sync_copy(s_hbm.at[pl.ds(rowbase, _NCHUNK)], sidx)
    pltpu.sync_copy(d_hbm.at[pl.ds(rowbase, _NCHUNK)], didx)
    plsc.subcore_barrier()

    def chunk_body(i, _):
        pltpu.sync_copy(q_hbm.at[pl.ds((rowbase + i) * _K, _K)], qbuf)
        pltpu.sync_copy(p_tab.at[sidx.at[i]], pbuf)

        def vec_body(r, _):
            for j in range(_H // 16):
                sl = pl.ds(j * 16, 16)
                x = pbuf[r, sl] + qbuf[r, sl]
                qbuf[r, sl] = jnp.maximum(x, 0.01 * x)
            return 0

        lax.fori_loop(0, _K, vec_body, 0)
        pltpu.sync_copy(qbuf, agg_tab.at[didx.at[i]], add=True)
        return 0

    lax.fori_loop(0, _NCHUNK, chunk_body, 0)
    plsc.subcore_barrier()
    pltpu.sync_copy(agg_tab.at[pl.ds(row0, _RPT)],
                    out_hbm.at[pl.ds(c * _N + row0, _RPT)])


_sc_layer = functools.partial(
    pl.kernel,
    out_type=jax.ShapeDtypeStruct((_NC * _N, _H), F32),
    mesh=plsc.VectorSubcoreMesh(core_axis_name="c", subcore_axis_name="s"),
    compiler_params=pltpu.CompilerParams(use_tc_tiling_on_sc=False),
    scratch_types=[
        pltpu.VMEM_SHARED((_N, _H), F32),
        pltpu.VMEM_SHARED((_N, _H), F32),
        pltpu.VMEM((_NCHUNK, _K), jnp.int32),
        pltpu.VMEM((_NCHUNK, _K), jnp.int32),
        pltpu.VMEM((_K, _H), F32),
        pltpu.VMEM((_K, _H), F32),
    ],
)(_sc_body)


def _upd_body(aggp_ref, h_ref, wu_ref, bu_ref, wm_ref, hn_ref, pn_ref):
    a = aggp_ref[...]
    agg = a[:_N] + a[_N:]
    t = jnp.dot(agg, wu_ref[...], preferred_element_type=F32) + bu_ref[...]
    hn = h_ref[...] + _leaky(t)
    hn_ref[...] = hn
    pn_ref[...] = jnp.dot(hn, wm_ref[...], preferred_element_type=F32)


def _upd_call(aggp, h, W_upd_l, b_upd_l, W_msg_next):
    return pl.pallas_call(
        _upd_body,
        out_shape=[
            jax.ShapeDtypeStruct((_N, _H), F32),
            jax.ShapeDtypeStruct((_N, _H), F32),
        ],
    )(aggp, h, W_upd_l, b_upd_l.reshape(1, _H), W_msg_next)


def _fin_body(aggp_ref, h_ref, wu_ref, bu_ref, w1_ref, b1_ref, w2_ref, b2_ref,
              o_ref):
    a = aggp_ref[...]
    agg = a[:_N] + a[_N:]
    t = jnp.dot(agg, wu_ref[...], preferred_element_type=F32) + bu_ref[...]
    hn = h_ref[...] + _leaky(t)
    g = jnp.sum(hn, axis=0, keepdims=True)
    g = _leaky(jnp.dot(g, w1_ref[...], preferred_element_type=F32) + b1_ref[...])
    o_ref[...] = jnp.dot(g, w2_ref[...], preferred_element_type=F32) + b2_ref[...]


def _fin_call(aggp, h, W_upd_l, b_upd_l, W_lin1, b_lin1, W_lin2, b_lin2):
    return pl.pallas_call(
        _fin_body,
        out_shape=jax.ShapeDtypeStruct((1, _T), F32),
    )(aggp, h, W_upd_l, b_upd_l.reshape(1, _H), W_lin1, b_lin1.reshape(1, _H),
      W_lin2, b_lin2.reshape(1, _T))


def kernel(node_feats, edge_feats, edge_index, W_node, b_node, W_edge, b_edge,
           W_msg, b_msg, W_upd, b_upd, W_lin1, b_lin1, W_lin2, b_lin2):
    src2d = edge_index[0].reshape(_E // _K, _K)
    dst2d = edge_index[1].reshape(_E // _K, _K)
    C_all = jnp.einsum('ij,ljk->lik', W_edge, W_msg)
    d_all = jnp.einsum('j,ljk->lk', b_edge, W_msg) + b_msg
    zeros_n = jnp.zeros((_N, _H), F32)

    Q = _q_call(edge_feats, C_all, d_all)
    h, p = _embed_call(node_feats, W_node, b_node, W_msg[0])
    for l in range(_L):
        aggp = _sc_layer(p, Q[l], src2d, dst2d, zeros_n)
        if l < _L - 1:
            h, p = _upd_call(aggp, h, W_upd[l], b_upd[l], W_msg[l + 1])
        else:
            out = _fin_call(aggp, h, W_upd[l], b_upd[l],
                            W_lin1, b_lin1, W_lin2, b_lin2)
    return out


# speedup vs baseline: 3.2348x; 1.0673x over previous
import functools

import jax
import jax.numpy as jnp
from jax import lax
from jax.experimental import pallas as pl
from jax.experimental.pallas import tpu as pltpu
from jax.experimental.pallas import tpu_sc as plsc

F32 = jnp.float32

_N = 10000
_E = 320000
_ND = 128
_ED = 16
_H = 64
_L = 4
_T = 1

_NC = 2
_NS = 16
_NW = _NC * _NS
_EPW = _E // _NW
_K = 80
_NCHUNK = _EPW // _K
_RPT = _N // _NS


def _leaky(x):
    return jnp.maximum(x, 0.01 * x)


_BE = 8000


def _q_body(ef_ref, c_ref, d_ref, q_ref):
    x = ef_ref[...]
    for l in range(_L):
        q_ref[l] = jnp.dot(x, c_ref[l], preferred_element_type=F32) + d_ref[l]


def _q_call(edge_feats, C_all, d_all):
    return pl.pallas_call(
        _q_body,
        grid=(_E // _BE,),
        in_specs=[
            pl.BlockSpec((_BE, _ED), lambda i: (i, 0)),
            pl.BlockSpec((_L, _ED, _H), lambda i: (0, 0, 0)),
            pl.BlockSpec((_L, _H), lambda i: (0, 0)),
        ],
        out_specs=pl.BlockSpec((_L, _BE, _H), lambda i: (0, i, 0)),
        out_shape=jax.ShapeDtypeStruct((_L, _E, _H), F32),
    )(edge_feats, C_all, d_all)


def _embed_body(nf_ref, wn_ref, bn_ref, wm0_ref, h_ref, p_ref):
    h = jnp.dot(nf_ref[...], wn_ref[...], preferred_element_type=F32) + bn_ref[...]
    h_ref[...] = h
    p_ref[...] = jnp.dot(h, wm0_ref[...], preferred_element_type=F32)


def _embed_call(node_feats, W_node, b_node, W_msg0):
    return pl.pallas_call(
        _embed_body,
        out_shape=[
            jax.ShapeDtypeStruct((_N, _H), F32),
            jax.ShapeDtypeStruct((_N, _H), F32),
        ],
    )(node_feats, W_node, b_node.reshape(1, _H), W_msg0)


_R = 5
_TPC = _NCHUNK // _R
_LAG = 2


def _sc_body(p_hbm, q_hbm, s_hbm, d_hbm, z_hbm, out_hbm,
             agg_tab, sidx, didx, qbuf, pbuf, *sems):
    qsem = sems[0:_R]
    psem = sems[_R:2 * _R]
    ssem = sems[2 * _R:3 * _R]
    c = lax.axis_index("c")
    s = lax.axis_index("s")
    row0 = s * _RPT
    pltpu.sync_copy(z_hbm.at[pl.ds(row0, _RPT)], agg_tab.at[pl.ds(row0, _RPT)])
    rowbase = (c * _NS + s) * _NCHUNK
    pltpu.sync_copy(s_hbm.at[pl.ds(rowbase, _NCHUNK)], sidx)
    pltpu.sync_copy(d_hbm.at[pl.ds(rowbase, _NCHUNK)], didx)
    plsc.subcore_barrier()

    def issue_loads(k, b):
        pltpu.async_copy(q_hbm.at[pl.ds((rowbase + k) * _K, _K)],
                         qbuf.at[b], qsem[b])
        pltpu.async_copy(p_hbm.at[sidx.at[k]], pbuf.at[b], psem[b])

    def wait_loads(k, b):
        pltpu.make_async_copy(q_hbm.at[pl.ds((rowbase + k) * _K, _K)],
                              qbuf.at[b], qsem[b]).wait()
        pltpu.make_async_copy(p_hbm.at[sidx.at[k]], pbuf.at[b], psem[b]).wait()

    def issue_scatter(k, b):
        pltpu.async_copy(pbuf.at[b], agg_tab.at[didx.at[k]], ssem[b], add=True)

    def wait_scatter(k, b):
        pltpu.make_async_copy(pbuf.at[b], agg_tab.at[didx.at[k]],
                              ssem[b]).wait()

    def compute(b):
        def vec_body(r, _):
            for j in range(_H // 16):
                sl = pl.ds(j * 16, 16)
                x = pbuf[b, r, sl] + qbuf[b, r, sl]
                pbuf[b, r, sl] = jnp.maximum(x, 0.01 * x)
            return 0

        lax.fori_loop(0, _K, vec_body, 0, unroll=2)

    def tail(k, b):
        j = k - _LAG
        bj = (b - _LAG) % _R
        wait_scatter(j, bj)
        issue_loads(j + _R, bj)

    def step(k, b, do_tail):
        wait_loads(k, b)
        compute(b)
        issue_scatter(k, b)
        if do_tail:
            tail(k, b)

    for b in range(_R):
        issue_loads(b, b)
    for b in range(_R):
        step(b, b, do_tail=(b >= _LAG))

    def outer(t, _):
        for b in range(_R):
            step(t * _R + b, b, do_tail=True)
        return 0

    lax.fori_loop(1, _TPC - 1, outer, 0)

    for b in range(_R):
        k = (_TPC - 1) * _R + b
        step(k, b, do_tail=(k - _LAG + _R < _NCHUNK))
        if not (k - _LAG + _R < _NCHUNK):
            wait_scatter(k - _LAG, (b - _LAG) % _R)
    for b in range(_R - _LAG, _R):
        wait_scatter((_TPC - 1) * _R + b, b)

    plsc.subcore_barrier()
    pltpu.sync_copy(agg_tab.at[pl.ds(row0, _RPT)],
                    out_hbm.at[pl.ds(c * _N + row0, _RPT)])


_sc_layer = functools.partial(
    pl.kernel,
    out_type=jax.ShapeDtypeStruct((_NC * _N, _H), F32),
    mesh=plsc.VectorSubcoreMesh(core_axis_name="c", subcore_axis_name="s"),
    compiler_params=pltpu.CompilerParams(use_tc_tiling_on_sc=False),
    scratch_types=[
        pltpu.VMEM_SHARED((_N, _H), F32),
        pltpu.VMEM((_NCHUNK, _K), jnp.int32),
        pltpu.VMEM((_NCHUNK, _K), jnp.int32),
        pltpu.VMEM((_R, _K, _H), F32),
        pltpu.VMEM((_R, _K, _H), F32),
    ] + [pltpu.SemaphoreType.DMA] * (3 * _R),
)(_sc_body)


def _upd_body(aggp_ref, h_ref, wu_ref, bu_ref, wm_ref, hn_ref, pn_ref):
    a = aggp_ref[...]
    agg = a[:_N] + a[_N:]
    t = jnp.dot(agg, wu_ref[...], preferred_element_type=F32) + bu_ref[...]
    hn = h_ref[...] + _leaky(t)
    hn_ref[...] = hn
    pn_ref[...] = jnp.dot(hn, wm_ref[...], preferred_element_type=F32)


def _upd_call(aggp, h, W_upd_l, b_upd_l, W_msg_next):
    return pl.pallas_call(
        _upd_body,
        out_shape=[
            jax.ShapeDtypeStruct((_N, _H), F32),
            jax.ShapeDtypeStruct((_N, _H), F32),
        ],
    )(aggp, h, W_upd_l, b_upd_l.reshape(1, _H), W_msg_next)


def _fin_body(aggp_ref, h_ref, wu_ref, bu_ref, w1_ref, b1_ref, w2_ref, b2_ref,
              o_ref):
    a = aggp_ref[...]
    agg = a[:_N] + a[_N:]
    t = jnp.dot(agg, wu_ref[...], preferred_element_type=F32) + bu_ref[...]
    hn = h_ref[...] + _leaky(t)
    g = jnp.sum(hn, axis=0, keepdims=True)
    g = _leaky(jnp.dot(g, w1_ref[...], preferred_element_type=F32) + b1_ref[...])
    o_ref[...] = jnp.dot(g, w2_ref[...], preferred_element_type=F32) + b2_ref[...]


def _fin_call(aggp, h, W_upd_l, b_upd_l, W_lin1, b_lin1, W_lin2, b_lin2):
    return pl.pallas_call(
        _fin_body,
        out_shape=jax.ShapeDtypeStruct((1, _T), F32),
    )(aggp, h, W_upd_l, b_upd_l.reshape(1, _H), W_lin1, b_lin1.reshape(1, _H),
      W_lin2, b_lin2.reshape(1, _T))


def kernel(node_feats, edge_feats, edge_index, W_node, b_node, W_edge, b_edge,
           W_msg, b_msg, W_upd, b_upd, W_lin1, b_lin1, W_lin2, b_lin2):
    src2d = edge_index[0].reshape(_E // _K, _K)
    dst2d = edge_index[1].reshape(_E // _K, _K)
    C_all = jnp.einsum('ij,ljk->lik', W_edge, W_msg)
    d_all = jnp.einsum('j,ljk->lk', b_edge, W_msg) + b_msg
    zeros_n = jnp.zeros((_N, _H), F32)

    Q = _q_call(edge_feats, C_all, d_all)
    h, p = _embed_call(node_feats, W_node, b_node, W_msg[0])
    for l in range(_L):
        aggp = _sc_layer(p, Q[l], src2d, dst2d, zeros_n)
        if l < _L - 1:
            h, p = _upd_call(aggp, h, W_upd[l], b_upd[l], W_msg[l + 1])
        else:
            out = _fin_call(aggp, h, W_upd[l], b_upd[l],
                            W_lin1, b_lin1, W_lin2, b_lin2)
    return out


# speedup vs baseline: 4.0187x; 1.2423x over previous
import functools

import jax
import jax.numpy as jnp
from jax import lax
from jax.experimental import pallas as pl
from jax.experimental.pallas import tpu as pltpu
from jax.experimental.pallas import tpu_sc as plsc

F32 = jnp.float32

_N = 10000
_E = 320000
_ND = 128
_ED = 16
_H = 64
_L = 4
_T = 1

_NC = 2
_NS = 16
_NW = _NC * _NS
_EPW = _E // _NW
_K = 80
_NCHUNK = _EPW // _K
_RPT = _N // _NS


def _leaky(x):
    return jnp.maximum(x, 0.01 * x)


_BE = 8000


def _q_body(ef_ref, c_ref, d_ref, q_ref):
    x = ef_ref[...]
    for l in range(_L):
        q_ref[l] = jnp.dot(x, c_ref[l], preferred_element_type=F32) + d_ref[l]


def _q_call(edge_feats, C_all, d_all):
    return pl.pallas_call(
        _q_body,
        grid=(_E // _BE,),
        in_specs=[
            pl.BlockSpec((_BE, _ED), lambda i: (i, 0)),
            pl.BlockSpec((_L, _ED, _H), lambda i: (0, 0, 0)),
            pl.BlockSpec((_L, _H), lambda i: (0, 0)),
        ],
        out_specs=pl.BlockSpec((_L, _BE, _H), lambda i: (0, i, 0)),
        out_shape=jax.ShapeDtypeStruct((_L, _E, _H), F32),
    )(edge_feats, C_all, d_all)


def _embed_body(nf_ref, wn_ref, bn_ref, wm0_ref, h_ref, p_ref):
    h = jnp.dot(nf_ref[...], wn_ref[...], preferred_element_type=F32) + bn_ref[...]
    h_ref[...] = h
    p_ref[...] = jnp.dot(h, wm0_ref[...], preferred_element_type=F32)


def _embed_call(node_feats, W_node, b_node, W_msg0):
    return pl.pallas_call(
        _embed_body,
        out_shape=[
            jax.ShapeDtypeStruct((_N, _H), F32),
            jax.ShapeDtypeStruct((_N, _H), F32),
        ],
    )(node_feats, W_node, b_node.reshape(1, _H), W_msg0)


_R = 5
_TPC = _NCHUNK // _R
_LAG = 2


def _sc_body(p_hbm, q_hbm, s_hbm, d_hbm, z_hbm, out_hbm,
             agg_tab, sidx, didx, qbuf, pbuf, *sems):
    qsem = sems[0:_R]
    psem = sems[_R:2 * _R]
    ssem = sems[2 * _R:3 * _R]
    c = lax.axis_index("c")
    s = lax.axis_index("s")
    row0 = s * _RPT
    pltpu.sync_copy(z_hbm.at[pl.ds(row0, _RPT)], agg_tab.at[pl.ds(row0, _RPT)])
    rowbase = (c * _NS + s) * _NCHUNK
    pltpu.sync_copy(s_hbm.at[pl.ds(rowbase, _NCHUNK)], sidx)
    pltpu.sync_copy(d_hbm.at[pl.ds(rowbase, _NCHUNK)], didx)
    plsc.subcore_barrier()

    def issue_loads(k, b):
        pltpu.async_copy(q_hbm.at[pl.ds((rowbase + k) * _K, _K)],
                         qbuf.at[b], qsem[b])
        pltpu.async_copy(p_hbm.at[sidx.at[k]], pbuf.at[b], psem[b])

    def wait_loads(k, b):
        pltpu.make_async_copy(q_hbm.at[pl.ds((rowbase + k) * _K, _K)],
                              qbuf.at[b], qsem[b]).wait()
        pltpu.make_async_copy(p_hbm.at[sidx.at[k]], pbuf.at[b], psem[b]).wait()

    def issue_scatter(k, b):
        pltpu.async_copy(pbuf.at[b], agg_tab.at[didx.at[k]], ssem[b], add=True)

    def wait_scatter(k, b):
        pltpu.make_async_copy(pbuf.at[b], agg_tab.at[didx.at[k]],
                              ssem[b]).wait()

    def compute(b):
        @plsc.parallel_loop(0, _K, 1, unroll=4)
        def _pl_body(r):
            for j in range(_H // 16):
                sl = pl.ds(j * 16, 16)
                x = pbuf[b, r, sl] + qbuf[b, r, sl]
                pbuf[b, r, sl] = jnp.maximum(x, 0.01 * x)

    def tail(k, b):
        j = k - _LAG
        bj = (b - _LAG) % _R
        wait_scatter(j, bj)
        issue_loads(j + _R, bj)

    def step(k, b, do_tail):
        wait_loads(k, b)
        compute(b)
        issue_scatter(k, b)
        if do_tail:
            tail(k, b)

    for b in range(_R):
        issue_loads(b, b)
    for b in range(_R):
        step(b, b, do_tail=(b >= _LAG))

    def outer(t, _):
        for b in range(_R):
            step(t * _R + b, b, do_tail=True)
        return 0

    lax.fori_loop(1, _TPC - 1, outer, 0)

    for b in range(_R):
        k = (_TPC - 1) * _R + b
        step(k, b, do_tail=(k - _LAG + _R < _NCHUNK))
        if not (k - _LAG + _R < _NCHUNK):
            wait_scatter(k - _LAG, (b - _LAG) % _R)
    for b in range(_R - _LAG, _R):
        wait_scatter((_TPC - 1) * _R + b, b)

    plsc.subcore_barrier()
    pltpu.sync_copy(agg_tab.at[pl.ds(row0, _RPT)],
                    out_hbm.at[pl.ds(c * _N + row0, _RPT)])


_sc_layer = functools.partial(
    pl.kernel,
    out_type=jax.ShapeDtypeStruct((_NC * _N, _H), F32),
    mesh=plsc.VectorSubcoreMesh(core_axis_name="c", subcore_axis_name="s"),
    compiler_params=pltpu.CompilerParams(use_tc_tiling_on_sc=False),
    scratch_types=[
        pltpu.VMEM_SHARED((_N, _H), F32),
        pltpu.VMEM((_NCHUNK, _K), jnp.int32),
        pltpu.VMEM((_NCHUNK, _K), jnp.int32),
        pltpu.VMEM((_R, _K, _H), F32),
        pltpu.VMEM((_R, _K, _H), F32),
    ] + [pltpu.SemaphoreType.DMA] * (3 * _R),
)(_sc_body)


def _upd_body(aggp_ref, h_ref, wu_ref, bu_ref, wm_ref, hn_ref, pn_ref):
    a = aggp_ref[...]
    agg = a[:_N] + a[_N:]
    t = jnp.dot(agg, wu_ref[...], preferred_element_type=F32) + bu_ref[...]
    hn = h_ref[...] + _leaky(t)
    hn_ref[...] = hn
    pn_ref[...] = jnp.dot(hn, wm_ref[...], preferred_element_type=F32)


def _upd_call(aggp, h, W_upd_l, b_upd_l, W_msg_next):
    return pl.pallas_call(
        _upd_body,
        out_shape=[
            jax.ShapeDtypeStruct((_N, _H), F32),
            jax.ShapeDtypeStruct((_N, _H), F32),
        ],
    )(aggp, h, W_upd_l, b_upd_l.reshape(1, _H), W_msg_next)


def _fin_body(aggp_ref, h_ref, wu_ref, bu_ref, w1_ref, b1_ref, w2_ref, b2_ref,
              o_ref):
    a = aggp_ref[...]
    agg = a[:_N] + a[_N:]
    t = jnp.dot(agg, wu_ref[...], preferred_element_type=F32) + bu_ref[...]
    hn = h_ref[...] + _leaky(t)
    g = jnp.sum(hn, axis=0, keepdims=True)
    g = _leaky(jnp.dot(g, w1_ref[...], preferred_element_type=F32) + b1_ref[...])
    o_ref[...] = jnp.dot(g, w2_ref[...], preferred_element_type=F32) + b2_ref[...]


def _fin_call(aggp, h, W_upd_l, b_upd_l, W_lin1, b_lin1, W_lin2, b_lin2):
    return pl.pallas_call(
        _fin_body,
        out_shape=jax.ShapeDtypeStruct((1, _T), F32),
    )(aggp, h, W_upd_l, b_upd_l.reshape(1, _H), W_lin1, b_lin1.reshape(1, _H),
      W_lin2, b_lin2.reshape(1, _T))


def kernel(node_feats, edge_feats, edge_index, W_node, b_node, W_edge, b_edge,
           W_msg, b_msg, W_upd, b_upd, W_lin1, b_lin1, W_lin2, b_lin2):
    src2d = edge_index[0].reshape(_E // _K, _K)
    dst2d = edge_index[1].reshape(_E // _K, _K)
    C_all = jnp.einsum('ij,ljk->lik', W_edge, W_msg)
    d_all = jnp.einsum('j,ljk->lk', b_edge, W_msg) + b_msg
    zeros_n = jnp.zeros((_N, _H), F32)

    Q = _q_call(edge_feats, C_all, d_all)
    h, p = _embed_call(node_feats, W_node, b_node, W_msg[0])
    for l in range(_L):
        aggp = _sc_layer(p, Q[l], src2d, dst2d, zeros_n)
        if l < _L - 1:
            h, p = _upd_call(aggp, h, W_upd[l], b_upd[l], W_msg[l + 1])
        else:
            out = _fin_call(aggp, h, W_upd[l], b_upd[l],
                            W_lin1, b_lin1, W_lin2, b_lin2)
    return out


# speedup vs baseline: 4.1839x; 1.0411x over previous
import functools

import jax
import jax.numpy as jnp
from jax import lax
from jax.experimental import pallas as pl
from jax.experimental.pallas import tpu as pltpu
from jax.experimental.pallas import tpu_sc as plsc

F32 = jnp.float32

_N = 10000
_E = 320000
_ND = 128
_ED = 16
_H = 64
_L = 4
_T = 1

_NC = 2
_NS = 16
_NW = _NC * _NS
_EPW = _E // _NW
_K = 40
_NCHUNK = _EPW // _K
_RPT = _N // _NS


def _leaky(x):
    return jnp.maximum(x, 0.01 * x)


_BE = 8000


def _q_body(ef_ref, c_ref, d_ref, q_ref):
    x = ef_ref[...]
    for l in range(_L):
        q_ref[l] = jnp.dot(x, c_ref[l], preferred_element_type=F32) + d_ref[l]


def _q_call(edge_feats, C_all, d_all):
    return pl.pallas_call(
        _q_body,
        grid=(_E // _BE,),
        in_specs=[
            pl.BlockSpec((_BE, _ED), lambda i: (i, 0)),
            pl.BlockSpec((_L, _ED, _H), lambda i: (0, 0, 0)),
            pl.BlockSpec((_L, _H), lambda i: (0, 0)),
        ],
        out_specs=pl.BlockSpec((_L, _BE, _H), lambda i: (0, i, 0)),
        out_shape=jax.ShapeDtypeStruct((_L, _E, _H), F32),
    )(edge_feats, C_all, d_all)


def _embed_body(nf_ref, wn_ref, bn_ref, wm0_ref, h_ref, p_ref):
    h = jnp.dot(nf_ref[...], wn_ref[...], preferred_element_type=F32) + bn_ref[...]
    h_ref[...] = h
    p_ref[...] = jnp.dot(h, wm0_ref[...], preferred_element_type=F32)


def _embed_call(node_feats, W_node, b_node, W_msg0):
    return pl.pallas_call(
        _embed_body,
        out_shape=[
            jax.ShapeDtypeStruct((_N, _H), F32),
            jax.ShapeDtypeStruct((_N, _H), F32),
        ],
    )(node_feats, W_node, b_node.reshape(1, _H), W_msg0)


_R = 5
_TPC = _NCHUNK // _R
_LAG = 2


def _sc_body(p_hbm, q_hbm, s_hbm, d_hbm, z_hbm, out_hbm,
             p_tab, agg_tab, sidx, didx, qbuf, pbuf, *sems):
    qsem = sems[0:_R]
    psem = sems[_R:2 * _R]
    ssem = sems[2 * _R:3 * _R]
    c = lax.axis_index("c")
    s = lax.axis_index("s")
    row0 = s * _RPT
    pltpu.sync_copy(p_hbm.at[pl.ds(row0, _RPT)], p_tab.at[pl.ds(row0, _RPT)])
    pltpu.sync_copy(z_hbm.at[pl.ds(row0, _RPT)], agg_tab.at[pl.ds(row0, _RPT)])
    rowbase = (c * _NS + s) * _NCHUNK
    pltpu.sync_copy(s_hbm.at[pl.ds(rowbase, _NCHUNK)], sidx)
    pltpu.sync_copy(d_hbm.at[pl.ds(rowbase, _NCHUNK)], didx)
    plsc.subcore_barrier()

    def issue_loads(k, b):
        pltpu.async_copy(q_hbm.at[pl.ds((rowbase + k) * _K, _K)],
                         qbuf.at[b], qsem[b])
        pltpu.async_copy(p_tab.at[sidx.at[k]], pbuf.at[b], psem[b])

    def wait_loads(k, b):
        pltpu.make_async_copy(q_hbm.at[pl.ds((rowbase + k) * _K, _K)],
                              qbuf.at[b], qsem[b]).wait()
        pltpu.make_async_copy(p_tab.at[sidx.at[k]], pbuf.at[b], psem[b]).wait()

    def issue_scatter(k, b):
        pltpu.async_copy(pbuf.at[b], agg_tab.at[didx.at[k]], ssem[b], add=True)

    def wait_scatter(k, b):
        pltpu.make_async_copy(pbuf.at[b], agg_tab.at[didx.at[k]],
                              ssem[b]).wait()

    def compute(b):
        @plsc.parallel_loop(0, _K, 1, unroll=4)
        def _pl_body(r):
            for j in range(_H // 16):
                sl = pl.ds(j * 16, 16)
                x = pbuf[b, r, sl] + qbuf[b, r, sl]
                pbuf[b, r, sl] = jnp.maximum(x, 0.01 * x)

    def tail(k, b):
        j = k - _LAG
        bj = (b - _LAG) % _R
        wait_scatter(j, bj)
        issue_loads(j + _R, bj)

    def step(k, b, do_tail):
        wait_loads(k, b)
        compute(b)
        issue_scatter(k, b)
        if do_tail:
            tail(k, b)

    for b in range(_R):
        issue_loads(b, b)
    for b in range(_R):
        step(b, b, do_tail=(b >= _LAG))

    def outer(t, _):
        for b in range(_R):
            step(t * _R + b, b, do_tail=True)
        return 0

    lax.fori_loop(1, _TPC - 1, outer, 0)

    for b in range(_R):
        k = (_TPC - 1) * _R + b
        step(k, b, do_tail=(k - _LAG + _R < _NCHUNK))
        if not (k - _LAG + _R < _NCHUNK):
            wait_scatter(k - _LAG, (b - _LAG) % _R)
    for b in range(_R - _LAG, _R):
        wait_scatter((_TPC - 1) * _R + b, b)

    plsc.subcore_barrier()
    pltpu.sync_copy(agg_tab.at[pl.ds(row0, _RPT)],
                    out_hbm.at[pl.ds(c * _N + row0, _RPT)])


_sc_layer = functools.partial(
    pl.kernel,
    out_type=jax.ShapeDtypeStruct((_NC * _N, _H), F32),
    mesh=plsc.VectorSubcoreMesh(core_axis_name="c", subcore_axis_name="s"),
    compiler_params=pltpu.CompilerParams(use_tc_tiling_on_sc=False),
    scratch_types=[
        pltpu.VMEM_SHARED((_N, _H), F32),
        pltpu.VMEM_SHARED((_N, _H), F32),
        pltpu.VMEM((_NCHUNK, _K), jnp.int32),
        pltpu.VMEM((_NCHUNK, _K), jnp.int32),
        pltpu.VMEM((_R, _K, _H), F32),
        pltpu.VMEM((_R, _K, _H), F32),
    ] + [pltpu.SemaphoreType.DMA] * (3 * _R),
)(_sc_body)


def _upd_body(aggp_ref, h_ref, wu_ref, bu_ref, wm_ref, hn_ref, pn_ref):
    a = aggp_ref[...]
    agg = a[:_N] + a[_N:]
    t = jnp.dot(agg, wu_ref[...], preferred_element_type=F32) + bu_ref[...]
    hn = h_ref[...] + _leaky(t)
    hn_ref[...] = hn
    pn_ref[...] = jnp.dot(hn, wm_ref[...], preferred_element_type=F32)


def _upd_call(aggp, h, W_upd_l, b_upd_l, W_msg_next):
    return pl.pallas_call(
        _upd_body,
        out_shape=[
            jax.ShapeDtypeStruct((_N, _H), F32),
            jax.ShapeDtypeStruct((_N, _H), F32),
        ],
    )(aggp, h, W_upd_l, b_upd_l.reshape(1, _H), W_msg_next)


def _fin_body(aggp_ref, h_ref, wu_ref, bu_ref, w1_ref, b1_ref, w2_ref, b2_ref,
              o_ref):
    a = aggp_ref[...]
    agg = a[:_N] + a[_N:]
    t = jnp.dot(agg, wu_ref[...], preferred_element_type=F32) + bu_ref[...]
    hn = h_ref[...] + _leaky(t)
    g = jnp.sum(hn, axis=0, keepdims=True)
    g = _leaky(jnp.dot(g, w1_ref[...], preferred_element_type=F32) + b1_ref[...])
    o_ref[...] = jnp.dot(g, w2_ref[...], preferred_element_type=F32) + b2_ref[...]


def _fin_call(aggp, h, W_upd_l, b_upd_l, W_lin1, b_lin1, W_lin2, b_lin2):
    return pl.pallas_call(
        _fin_body,
        out_shape=jax.ShapeDtypeStruct((1, _T), F32),
    )(aggp, h, W_upd_l, b_upd_l.reshape(1, _H), W_lin1, b_lin1.reshape(1, _H),
      W_lin2, b_lin2.reshape(1, _T))


def kernel(node_feats, edge_feats, edge_index, W_node, b_node, W_edge, b_edge,
           W_msg, b_msg, W_upd, b_upd, W_lin1, b_lin1, W_lin2, b_lin2):
    src2d = edge_index[0].reshape(_E // _K, _K)
    dst2d = edge_index[1].reshape(_E // _K, _K)
    C_all = jnp.einsum('ij,ljk->lik', W_edge, W_msg)
    d_all = jnp.einsum('j,ljk->lk', b_edge, W_msg) + b_msg
    zeros_n = jnp.zeros((_N, _H), F32)

    Q = _q_call(edge_feats, C_all, d_all)
    h, p = _embed_call(node_feats, W_node, b_node, W_msg[0])
    for l in range(_L):
        aggp = _sc_layer(p, Q[l], src2d, dst2d, zeros_n)
        if l < _L - 1:
            h, p = _upd_call(aggp, h, W_upd[l], b_upd[l], W_msg[l + 1])
        else:
            out = _fin_call(aggp, h, W_upd[l], b_upd[l],
                            W_lin1, b_lin1, W_lin2, b_lin2)
    return out


# speedup vs baseline: 7.8015x; 1.8647x over previous
import functools

import jax
import jax.numpy as jnp
from jax import lax
from jax.experimental import pallas as pl
from jax.experimental.pallas import tpu as pltpu
from jax.experimental.pallas import tpu_sc as plsc

F32 = jnp.float32

_N = 10000
_E = 320000
_ND = 128
_ED = 16
_H = 64
_L = 4
_T = 1

_NC = 2
_NS = 16
_NW = _NC * _NS
_EPW = _E // _NW
_K = 40
_NCHUNK = _EPW // _K
_RPT = _N // _NS


def _leaky(x):
    return jnp.maximum(x, 0.01 * x)


_E2 = _E // 2
_BE2 = 8000


def _q_body(ef_ref, c_ref, d_ref, q_ref):
    x = ef_ref[...]
    for l in range(_L):
        q_ref[l] = jnp.dot(x, c_ref[l], preferred_element_type=F32) + d_ref[l]


def _q_call(ef2, C_blk, d2):
    return pl.pallas_call(
        _q_body,
        grid=(_E2 // _BE2,),
        in_specs=[
            pl.BlockSpec((_BE2, 2 * _ED), lambda i: (i, 0)),
            pl.BlockSpec((_L, 2 * _ED, 128), lambda i: (0, 0, 0)),
            pl.BlockSpec((_L, 128), lambda i: (0, 0)),
        ],
        out_specs=pl.BlockSpec((_L, _BE2, 128), lambda i: (0, i, 0)),
        out_shape=jax.ShapeDtypeStruct((_L, _E2, 128), F32),
    )(ef2, C_blk, d2)


def _embed_body(nf_ref, wn_ref, bn_ref, wm0_ref, h_ref, p_ref):
    h = jnp.dot(nf_ref[...], wn_ref[...], preferred_element_type=F32) + bn_ref[...]
    h_ref[...] = h
    p_ref[...] = jnp.dot(h, wm0_ref[...], preferred_element_type=F32)


def _embed_call(node_feats, W_node, b_node, W_msg0):
    return pl.pallas_call(
        _embed_body,
        out_shape=[
            jax.ShapeDtypeStruct((_N, _H), F32),
            jax.ShapeDtypeStruct((_N, _H), F32),
        ],
    )(node_feats, W_node, b_node.reshape(1, _H), W_msg0)


_R = 5
_TPC = _NCHUNK // _R
_LAG = 2


_KH = _K // 2


def _sc_body(l, p_hbm, q_hbm, s_hbm, d_hbm, z_hbm, out_hbm,
             p_tab, agg_tab, sidx, didx, qbuf, pbuf, *sems):
    qsem = sems[0:_R]
    psem = sems[_R:2 * _R]
    ssem = sems[2 * _R:3 * _R]
    c = lax.axis_index("c")
    s = lax.axis_index("s")
    row0 = s * _RPT
    pltpu.sync_copy(p_hbm.at[pl.ds(row0, _RPT)], p_tab.at[pl.ds(row0, _RPT)])
    pltpu.sync_copy(z_hbm.at[pl.ds(row0, _RPT)], agg_tab.at[pl.ds(row0, _RPT)])
    rowbase = (c * _NS + s) * _NCHUNK
    pltpu.sync_copy(s_hbm.at[pl.ds(rowbase, _NCHUNK)], sidx)
    pltpu.sync_copy(d_hbm.at[pl.ds(rowbase, _NCHUNK)], didx)
    plsc.subcore_barrier()

    def issue_loads(k, b):
        pltpu.async_copy(q_hbm.at[l, pl.ds((rowbase + k) * _KH, _KH)],
                         qbuf.at[b], qsem[b])
        pltpu.async_copy(p_tab.at[sidx.at[k]], pbuf.at[b], psem[b])

    def wait_loads(k, b):
        pltpu.make_async_copy(q_hbm.at[l, pl.ds((rowbase + k) * _KH, _KH)],
                              qbuf.at[b], qsem[b]).wait()
        pltpu.make_async_copy(p_tab.at[sidx.at[k]], pbuf.at[b], psem[b]).wait()

    def issue_scatter(k, b):
        pltpu.async_copy(pbuf.at[b], agg_tab.at[didx.at[k]], ssem[b], add=True)

    def wait_scatter(k, b):
        pltpu.make_async_copy(pbuf.at[b], agg_tab.at[didx.at[k]],
                              ssem[b]).wait()

    def compute(b):
        @plsc.parallel_loop(0, _KH, 1, unroll=4)
        def _pl_body(r):
            for j in range(128 // 16):
                x = (pbuf[b, 2 * r + j // 4, pl.ds((j % 4) * 16, 16)]
                     + qbuf[b, r, pl.ds(j * 16, 16)])
                pbuf[b, 2 * r + j // 4, pl.ds((j % 4) * 16, 16)] = (
                    jnp.maximum(x, 0.01 * x))

    def tail(k, b):
        j = k - _LAG
        bj = (b - _LAG) % _R
        wait_scatter(j, bj)
        issue_loads(j + _R, bj)

    def step(k, b, do_tail):
        wait_loads(k, b)
        compute(b)
        issue_scatter(k, b)
        if do_tail:
            tail(k, b)

    for b in range(_R):
        issue_loads(b, b)
    for b in range(_R):
        step(b, b, do_tail=(b >= _LAG))

    def outer(t, _):
        for b in range(_R):
            step(t * _R + b, b, do_tail=True)
        return 0

    lax.fori_loop(1, _TPC - 1, outer, 0)

    for b in range(_R):
        k = (_TPC - 1) * _R + b
        step(k, b, do_tail=(k - _LAG + _R < _NCHUNK))
        if not (k - _LAG + _R < _NCHUNK):
            wait_scatter(k - _LAG, (b - _LAG) % _R)
    for b in range(_R - _LAG, _R):
        wait_scatter((_TPC - 1) * _R + b, b)

    plsc.subcore_barrier()
    pltpu.sync_copy(agg_tab.at[pl.ds(row0, _RPT)],
                    out_hbm.at[pl.ds(c * _N + row0, _RPT)])


_sc_layers = [
    functools.partial(
        pl.kernel,
        out_type=jax.ShapeDtypeStruct((_NC * _N, _H), F32),
        mesh=plsc.VectorSubcoreMesh(core_axis_name="c", subcore_axis_name="s"),
        compiler_params=pltpu.CompilerParams(use_tc_tiling_on_sc=False),
        scratch_types=[
            pltpu.VMEM_SHARED((_N, _H), F32),
            pltpu.VMEM_SHARED((_N, _H), F32),
            pltpu.VMEM((_NCHUNK, _K), jnp.int32),
            pltpu.VMEM((_NCHUNK, _K), jnp.int32),
            pltpu.VMEM((_R, _KH, 128), F32),
            pltpu.VMEM((_R, _K, _H), F32),
        ] + [pltpu.SemaphoreType.DMA] * (3 * _R),
    )(functools.partial(_sc_body, _lyr))
    for _lyr in range(_L)
]


def _upd_body(aggp_ref, h_ref, wu_ref, bu_ref, wm_ref, hn_ref, pn_ref):
    a = aggp_ref[...]
    agg = a[:_N] + a[_N:]
    t = jnp.dot(agg, wu_ref[...], preferred_element_type=F32) + bu_ref[...]
    hn = h_ref[...] + _leaky(t)
    hn_ref[...] = hn
    pn_ref[...] = jnp.dot(hn, wm_ref[...], preferred_element_type=F32)


def _upd_call(aggp, h, W_upd_l, b_upd_l, W_msg_next):
    return pl.pallas_call(
        _upd_body,
        out_shape=[
            jax.ShapeDtypeStruct((_N, _H), F32),
            jax.ShapeDtypeStruct((_N, _H), F32),
        ],
    )(aggp, h, W_upd_l, b_upd_l.reshape(1, _H), W_msg_next)


def _fin_body(aggp_ref, h_ref, wu_ref, bu_ref, w1_ref, b1_ref, w2_ref, b2_ref,
              o_ref):
    a = aggp_ref[...]
    agg = a[:_N] + a[_N:]
    t = jnp.dot(agg, wu_ref[...], preferred_element_type=F32) + bu_ref[...]
    hn = h_ref[...] + _leaky(t)
    g = jnp.sum(hn, axis=0, keepdims=True)
    g = _leaky(jnp.dot(g, w1_ref[...], preferred_element_type=F32) + b1_ref[...])
    o_ref[...] = jnp.dot(g, w2_ref[...], preferred_element_type=F32) + b2_ref[...]


def _fin_call(aggp, h, W_upd_l, b_upd_l, W_lin1, b_lin1, W_lin2, b_lin2):
    return pl.pallas_call(
        _fin_body,
        out_shape=jax.ShapeDtypeStruct((1, _T), F32),
    )(aggp, h, W_upd_l, b_upd_l.reshape(1, _H), W_lin1, b_lin1.reshape(1, _H),
      W_lin2, b_lin2.reshape(1, _T))


def kernel(node_feats, edge_feats, edge_index, W_node, b_node, W_edge, b_edge,
           W_msg, b_msg, W_upd, b_upd, W_lin1, b_lin1, W_lin2, b_lin2):
    src2d = edge_index[0].reshape(_E // _K, _K)
    dst2d = edge_index[1].reshape(_E // _K, _K)
    C_all = jnp.einsum('ij,ljk->lik', W_edge, W_msg)
    d_all = jnp.einsum('j,ljk->lk', b_edge, W_msg) + b_msg
    C_blk = jnp.zeros((_L, 2 * _ED, 128), F32)
    C_blk = C_blk.at[:, :_ED, :_H].set(C_all).at[:, _ED:, _H:].set(C_all)
    d2 = jnp.concatenate([d_all, d_all], axis=-1)
    ef2 = edge_feats.reshape(_E2, 2 * _ED)
    zeros_n = jnp.zeros((_N, _H), F32)

    Q = _q_call(ef2, C_blk, d2)
    h, p = _embed_call(node_feats, W_node, b_node, W_msg[0])
    for l in range(_L):
        aggp = _sc_layers[l](p, Q, src2d, dst2d, zeros_n)
        if l < _L - 1:
            h, p = _upd_call(aggp, h, W_upd[l], b_upd[l], W_msg[l + 1])
        else:
            out = _fin_call(aggp, h, W_upd[l], b_upd[l],
                            W_lin1, b_lin1, W_lin2, b_lin2)
    return out


# speedup vs baseline: 7.9279x; 1.0162x over previous
import functools

import jax
import jax.numpy as jnp
from jax import lax
from jax.experimental import pallas as pl
from jax.experimental.pallas import tpu as pltpu
from jax.experimental.pallas import tpu_sc as plsc

F32 = jnp.float32

_N = 10000
_E = 320000
_ND = 128
_ED = 16
_H = 64
_L = 4
_T = 1

_NC = 2
_NS = 16
_NW = _NC * _NS
_EPW = _E // _NW
_K = 80
_NCHUNK = _EPW // _K
_RPT = _N // _NS


def _leaky(x):
    return jnp.maximum(x, 0.01 * x)


_E2 = _E // 2
_BE2 = 8000


def _q_body(ef_ref, c_ref, d_ref, q_ref):
    x = ef_ref[...]
    for l in range(_L):
        q_ref[l] = jnp.dot(x, c_ref[l], preferred_element_type=F32) + d_ref[l]


def _q_call(ef2, C_blk, d2):
    return pl.pallas_call(
        _q_body,
        grid=(_E2 // _BE2,),
        in_specs=[
            pl.BlockSpec((_BE2, 2 * _ED), lambda i: (i, 0)),
            pl.BlockSpec((_L, 2 * _ED, 128), lambda i: (0, 0, 0)),
            pl.BlockSpec((_L, 128), lambda i: (0, 0)),
        ],
        out_specs=pl.BlockSpec((_L, _BE2, 128), lambda i: (0, i, 0)),
        out_shape=jax.ShapeDtypeStruct((_L, _E2, 128), F32),
    )(ef2, C_blk, d2)


def _embed_body(nf_ref, wn_ref, bn_ref, wm0_ref, h_ref, p_ref):
    h = jnp.dot(nf_ref[...], wn_ref[...], preferred_element_type=F32) + bn_ref[...]
    h_ref[...] = h
    p_ref[...] = jnp.dot(h, wm0_ref[...], preferred_element_type=F32)


def _embed_call(node_feats, W_node, b_node, W_msg0):
    return pl.pallas_call(
        _embed_body,
        out_shape=[
            jax.ShapeDtypeStruct((_N, _H), F32),
            jax.ShapeDtypeStruct((_N, _H), F32),
        ],
    )(node_feats, W_node, b_node.reshape(1, _H), W_msg0)


_R = 5
_TPC = _NCHUNK // _R
_LAG = 2


_KH = _K // 2


def _sc_body(l, p_hbm, q_hbm, s_hbm, d_hbm, z_hbm, out_hbm,
             agg_tab, sidx, didx, qbuf, pbuf, *sems):
    qsem = sems[0:_R]
    psem = sems[_R:2 * _R]
    ssem = sems[2 * _R:3 * _R]
    c = lax.axis_index("c")
    s = lax.axis_index("s")
    row0 = s * _RPT
    pltpu.sync_copy(z_hbm.at[pl.ds(row0, _RPT)], agg_tab.at[pl.ds(row0, _RPT)])
    rowbase = (c * _NS + s) * _NCHUNK
    pltpu.sync_copy(s_hbm.at[pl.ds(rowbase, _NCHUNK)], sidx)
    pltpu.sync_copy(d_hbm.at[pl.ds(rowbase, _NCHUNK)], didx)
    plsc.subcore_barrier()

    def issue_loads(k, b):
        pltpu.async_copy(q_hbm.at[l, pl.ds((rowbase + k) * _KH, _KH)],
                         qbuf.at[b], qsem[b])
        pltpu.async_copy(p_hbm.at[sidx.at[k]], pbuf.at[b], psem[b])

    def wait_loads(k, b):
        pltpu.make_async_copy(q_hbm.at[l, pl.ds((rowbase + k) * _KH, _KH)],
                              qbuf.at[b], qsem[b]).wait()
        pltpu.make_async_copy(p_hbm.at[sidx.at[k]], pbuf.at[b], psem[b]).wait()

    def issue_scatter(k, b):
        pltpu.async_copy(pbuf.at[b], agg_tab.at[didx.at[k]], ssem[b], add=True)

    def wait_scatter(k, b):
        pltpu.make_async_copy(pbuf.at[b], agg_tab.at[didx.at[k]],
                              ssem[b]).wait()

    def compute(b):
        @plsc.parallel_loop(0, _KH, 1, unroll=4)
        def _pl_body(r):
            for j in range(128 // 16):
                x = (pbuf[b, 2 * r + j // 4, pl.ds((j % 4) * 16, 16)]
                     + qbuf[b, r, pl.ds(j * 16, 16)])
                pbuf[b, 2 * r + j // 4, pl.ds((j % 4) * 16, 16)] = (
                    jnp.maximum(x, 0.01 * x))

    def tail(k, b):
        j = k - _LAG
        bj = (b - _LAG) % _R
        wait_scatter(j, bj)
        issue_loads(j + _R, bj)

    def step(k, b, do_tail):
        wait_loads(k, b)
        compute(b)
        issue_scatter(k, b)
        if do_tail:
            tail(k, b)

    for b in range(_R):
        issue_loads(b, b)
    for b in range(_R):
        step(b, b, do_tail=(b >= _LAG))

    def outer(t, _):
        for b in range(_R):
            step(t * _R + b, b, do_tail=True)
        return 0

    lax.fori_loop(1, _TPC - 1, outer, 0)

    for b in range(_R):
        k = (_TPC - 1) * _R + b
        step(k, b, do_tail=(k - _LAG + _R < _NCHUNK))
        if not (k - _LAG + _R < _NCHUNK):
            wait_scatter(k - _LAG, (b - _LAG) % _R)
    for b in range(_R - _LAG, _R):
        wait_scatter((_TPC - 1) * _R + b, b)

    plsc.subcore_barrier()
    pltpu.sync_copy(agg_tab.at[pl.ds(row0, _RPT)],
                    out_hbm.at[pl.ds(c * _N + row0, _RPT)])


_sc_layers = [
    functools.partial(
        pl.kernel,
        out_type=jax.ShapeDtypeStruct((_NC * _N, _H), F32),
        mesh=plsc.VectorSubcoreMesh(core_axis_name="c", subcore_axis_name="s"),
        compiler_params=pltpu.CompilerParams(use_tc_tiling_on_sc=False),
        scratch_types=[
            pltpu.VMEM_SHARED((_N, _H), F32),
            pltpu.VMEM((_NCHUNK, _K), jnp.int32),
            pltpu.VMEM((_NCHUNK, _K), jnp.int32),
            pltpu.VMEM((_R, _KH, 128), F32),
            pltpu.VMEM((_R, _K, _H), F32),
        ] + [pltpu.SemaphoreType.DMA] * (3 * _R),
    )(functools.partial(_sc_body, _lyr))
    for _lyr in range(_L)
]


def _upd_body(aggp_ref, h_ref, wu_ref, bu_ref, wm_ref, hn_ref, pn_ref):
    a = aggp_ref[...]
    agg = a[:_N] + a[_N:]
    t = jnp.dot(agg, wu_ref[...], preferred_element_type=F32) + bu_ref[...]
    hn = h_ref[...] + _leaky(t)
    hn_ref[...] = hn
    pn_ref[...] = jnp.dot(hn, wm_ref[...], preferred_element_type=F32)


def _upd_call(aggp, h, W_upd_l, b_upd_l, W_msg_next):
    return pl.pallas_call(
        _upd_body,
        out_shape=[
            jax.ShapeDtypeStruct((_N, _H), F32),
            jax.ShapeDtypeStruct((_N, _H), F32),
        ],
    )(aggp, h, W_upd_l, b_upd_l.reshape(1, _H), W_msg_next)


def _fin_body(aggp_ref, h_ref, wu_ref, bu_ref, w1_ref, b1_ref, w2_ref, b2_ref,
              o_ref):
    a = aggp_ref[...]
    agg = a[:_N] + a[_N:]
    t = jnp.dot(agg, wu_ref[...], preferred_element_type=F32) + bu_ref[...]
    hn = h_ref[...] + _leaky(t)
    g = jnp.sum(hn, axis=0, keepdims=True)
    g = _leaky(jnp.dot(g, w1_ref[...], preferred_element_type=F32) + b1_ref[...])
    o_ref[...] = jnp.dot(g, w2_ref[...], preferred_element_type=F32) + b2_ref[...]


def _fin_call(aggp, h, W_upd_l, b_upd_l, W_lin1, b_lin1, W_lin2, b_lin2):
    return pl.pallas_call(
        _fin_body,
        out_shape=jax.ShapeDtypeStruct((1, _T), F32),
    )(aggp, h, W_upd_l, b_upd_l.reshape(1, _H), W_lin1, b_lin1.reshape(1, _H),
      W_lin2, b_lin2.reshape(1, _T))


def kernel(node_feats, edge_feats, edge_index, W_node, b_node, W_edge, b_edge,
           W_msg, b_msg, W_upd, b_upd, W_lin1, b_lin1, W_lin2, b_lin2):
    src2d = edge_index[0].reshape(_E // _K, _K)
    dst2d = edge_index[1].reshape(_E // _K, _K)
    C_all = jnp.einsum('ij,ljk->lik', W_edge, W_msg)
    d_all = jnp.einsum('j,ljk->lk', b_edge, W_msg) + b_msg
    C_blk = jnp.zeros((_L, 2 * _ED, 128), F32)
    C_blk = C_blk.at[:, :_ED, :_H].set(C_all).at[:, _ED:, _H:].set(C_all)
    d2 = jnp.concatenate([d_all, d_all], axis=-1)
    ef2 = edge_feats.reshape(_E2, 2 * _ED)
    zeros_n = jnp.zeros((_N, _H), F32)

    Q = _q_call(ef2, C_blk, d2)
    h, p = _embed_call(node_feats, W_node, b_node, W_msg[0])
    for l in range(_L):
        aggp = _sc_layers[l](p, Q, src2d, dst2d, zeros_n)
        if l < _L - 1:
            h, p = _upd_call(aggp, h, W_upd[l], b_upd[l], W_msg[l + 1])
        else:
            out = _fin_call(aggp, h, W_upd[l], b_upd[l],
                            W_lin1, b_lin1, W_lin2, b_lin2)
    return out


# speedup vs baseline: 8.5460x; 1.0780x over previous
import functools

import jax
import jax.numpy as jnp
from jax import lax
from jax.experimental import pallas as pl
from jax.experimental.pallas import tpu as pltpu
from jax.experimental.pallas import tpu_sc as plsc

F32 = jnp.float32

_N = 10000
_E = 320000
_ND = 128
_ED = 16
_H = 64
_L = 4
_T = 1

_NC = 2
_NS = 16
_NW = _NC * _NS
_EPW = _E // _NW
_K = 80
_NCHUNK = _EPW // _K
_RPT = _N // _NS


def _leaky(x):
    return jnp.maximum(x, 0.01 * x)


_E8 = _E // 8
_BE8 = 4000


def _pack2(lo, hi):
    ulo = jax.lax.bitcast_convert_type(
        lo.astype(jnp.bfloat16), jnp.uint16).astype(jnp.uint32)
    uhi = jax.lax.bitcast_convert_type(
        hi.astype(jnp.bfloat16), jnp.uint16).astype(jnp.uint32)
    return ulo | (uhi << 16)


def _q_body(ef_ref, clo_ref, chi_ref, dlo_ref, dhi_ref, q_ref):
    x = ef_ref[...]
    for l in range(_L):
        lo = jnp.dot(x, clo_ref[l], preferred_element_type=F32) + dlo_ref[l]
        hi = jnp.dot(x, chi_ref[l], preferred_element_type=F32) + dhi_ref[l]
        q_ref[l] = _pack2(lo, hi)


def _q_call(ef8, C_lo, C_hi, d_lo, d_hi):
    return pl.pallas_call(
        _q_body,
        grid=(_E8 // _BE8,),
        in_specs=[
            pl.BlockSpec((_BE8, 128), lambda i: (i, 0)),
            pl.BlockSpec((_L, 128, 256), lambda i: (0, 0, 0)),
            pl.BlockSpec((_L, 128, 256), lambda i: (0, 0, 0)),
            pl.BlockSpec((_L, 256), lambda i: (0, 0)),
            pl.BlockSpec((_L, 256), lambda i: (0, 0)),
        ],
        out_specs=pl.BlockSpec((_L, _BE8, 256), lambda i: (0, i, 0)),
        out_shape=jax.ShapeDtypeStruct((_L, _E8, 256), jnp.uint32),
    )(ef8, C_lo, C_hi, d_lo, d_hi)


def _embed_body(nf_ref, wn_ref, bn_ref, wm0_ref, h_ref, p_ref):
    h = jnp.dot(nf_ref[...], wn_ref[...], preferred_element_type=F32) + bn_ref[...]
    h_ref[...] = h
    wm = wm0_ref[...]
    p_ref[...] = _pack2(jnp.dot(h, wm[:, :32], preferred_element_type=F32),
                        jnp.dot(h, wm[:, 32:], preferred_element_type=F32))


def _embed_call(node_feats, W_node, b_node, W_msg0):
    return pl.pallas_call(
        _embed_body,
        out_shape=[
            jax.ShapeDtypeStruct((_N, _H), F32),
            jax.ShapeDtypeStruct((_N, _H // 2), jnp.uint32),
        ],
    )(node_feats, W_node, b_node.reshape(1, _H), W_msg0)


_R = 5
_TPC = _NCHUNK // _R
_LAG = 2


_KH = _K // 8


def _sc_body(l, p_hbm, q_hbm, s_hbm, d_hbm, z_hbm, out_hbm,
             agg_tab, sidx, didx, qbuf, pbuf, mbuf, *sems):
    qsem = sems[0:_R]
    psem = sems[_R:2 * _R]
    ssem = sems[2 * _R:3 * _R]
    c = lax.axis_index("c")
    s = lax.axis_index("s")
    row0 = s * _RPT
    pltpu.sync_copy(z_hbm.at[pl.ds(row0, _RPT)], agg_tab.at[pl.ds(row0, _RPT)])
    rowbase = (c * _NS + s) * _NCHUNK
    pltpu.sync_copy(s_hbm.at[pl.ds(rowbase, _NCHUNK)], sidx)
    pltpu.sync_copy(d_hbm.at[pl.ds(rowbase, _NCHUNK)], didx)
    plsc.subcore_barrier()

    def issue_loads(k, b):
        pltpu.async_copy(q_hbm.at[l, pl.ds((rowbase + k) * _KH, _KH)],
                         qbuf.at[b], qsem[b])
        pltpu.async_copy(p_hbm.at[sidx.at[k]], pbuf.at[b], psem[b])

    def wait_loads(k, b):
        pltpu.make_async_copy(q_hbm.at[l, pl.ds((rowbase + k) * _KH, _KH)],
                              qbuf.at[b], qsem[b]).wait()
        pltpu.make_async_copy(p_hbm.at[sidx.at[k]], pbuf.at[b], psem[b]).wait()

    def issue_scatter(k, b):
        pltpu.async_copy(mbuf.at[b], agg_tab.at[didx.at[k]], ssem[b], add=True)

    def wait_scatter(k, b):
        pltpu.make_async_copy(mbuf.at[b], agg_tab.at[didx.at[k]],
                              ssem[b]).wait()

    def compute(b):
        @plsc.parallel_loop(0, _K, 1, unroll=4)
        def _pl_body(r):
            for half in range(2):
                qw = qbuf[b, r // 8, pl.ds((r % 8) * 32 + half * 16, 16)]
                pw = pbuf[b, r, pl.ds(half * 16, 16)]
                qa, qb_ = plsc.unpack(plsc.bitcast(qw, jnp.bfloat16),
                                      format=plsc.PackFormat.INTERLEAVED)
                pa, pb_ = plsc.unpack(plsc.bitcast(pw, jnp.bfloat16),
                                      format=plsc.PackFormat.INTERLEAVED)
                x0 = qa + pa
                x1 = qb_ + pb_
                mbuf[b, r, pl.ds(half * 16, 16)] = jnp.maximum(x0, 0.01 * x0)
                mbuf[b, r, pl.ds(half * 16 + 32, 16)] = (
                    jnp.maximum(x1, 0.01 * x1))

    def tail(k, b):
        j = k - _LAG
        bj = (b - _LAG) % _R
        wait_scatter(j, bj)
        issue_loads(j + _R, bj)

    def step(k, b, do_tail):
        wait_loads(k, b)
        compute(b)
        issue_scatter(k, b)
        if do_tail:
            tail(k, b)

    for b in range(_R):
        issue_loads(b, b)
    for b in range(_R):
        step(b, b, do_tail=(b >= _LAG))

    def outer(t, _):
        for b in range(_R):
            step(t * _R + b, b, do_tail=True)
        return 0

    lax.fori_loop(1, _TPC - 1, outer, 0)

    for b in range(_R):
        k = (_TPC - 1) * _R + b
        step(k, b, do_tail=(k - _LAG + _R < _NCHUNK))
        if not (k - _LAG + _R < _NCHUNK):
            wait_scatter(k - _LAG, (b - _LAG) % _R)
    for b in range(_R - _LAG, _R):
        wait_scatter((_TPC - 1) * _R + b, b)

    plsc.subcore_barrier()
    pltpu.sync_copy(agg_tab.at[pl.ds(row0, _RPT)],
                    out_hbm.at[pl.ds(c * _N + row0, _RPT)])


_sc_layers = [
    functools.partial(
        pl.kernel,
        out_type=jax.ShapeDtypeStruct((_NC * _N, _H), F32),
        mesh=plsc.VectorSubcoreMesh(core_axis_name="c", subcore_axis_name="s"),
        compiler_params=pltpu.CompilerParams(use_tc_tiling_on_sc=False,
                                             needs_layout_passes=False),
        scratch_types=[
            pltpu.VMEM_SHARED((_N, _H), F32),
            pltpu.VMEM((_NCHUNK, _K), jnp.int32),
            pltpu.VMEM((_NCHUNK, _K), jnp.int32),
            pltpu.VMEM((_R, _KH, 256), jnp.uint32),
            pltpu.VMEM((_R, _K, _H // 2), jnp.uint32),
            pltpu.VMEM((_R, _K, _H), F32),
        ] + [pltpu.SemaphoreType.DMA] * (3 * _R),
    )(functools.partial(_sc_body, _lyr))
    for _lyr in range(_L)
]


def _upd_body(aggp_ref, h_ref, wu_ref, bu_ref, wm_ref, hn_ref, pn_ref):
    a = aggp_ref[...]
    agg = a[:_N] + a[_N:]
    t = jnp.dot(agg, wu_ref[...], preferred_element_type=F32) + bu_ref[...]
    hn = h_ref[...] + _leaky(t)
    hn_ref[...] = hn
    wm = wm_ref[...]
    pn_ref[...] = _pack2(jnp.dot(hn, wm[:, :32], preferred_element_type=F32),
                         jnp.dot(hn, wm[:, 32:], preferred_element_type=F32))


def _upd_call(aggp, h, W_upd_l, b_upd_l, W_msg_next):
    return pl.pallas_call(
        _upd_body,
        out_shape=[
            jax.ShapeDtypeStruct((_N, _H), F32),
            jax.ShapeDtypeStruct((_N, _H // 2), jnp.uint32),
        ],
    )(aggp, h, W_upd_l, b_upd_l.reshape(1, _H), W_msg_next)


def _fin_body(aggp_ref, h_ref, wu_ref, bu_ref, w1_ref, b1_ref, w2_ref, b2_ref,
              o_ref):
    a = aggp_ref[...]
    agg = a[:_N] + a[_N:]
    t = jnp.dot(agg, wu_ref[...], preferred_element_type=F32) + bu_ref[...]
    hn = h_ref[...] + _leaky(t)
    g = jnp.sum(hn, axis=0, keepdims=True)
    g = _leaky(jnp.dot(g, w1_ref[...], preferred_element_type=F32) + b1_ref[...])
    o_ref[...] = jnp.dot(g, w2_ref[...], preferred_element_type=F32) + b2_ref[...]


def _fin_call(aggp, h, W_upd_l, b_upd_l, W_lin1, b_lin1, W_lin2, b_lin2):
    return pl.pallas_call(
        _fin_body,
        out_shape=jax.ShapeDtypeStruct((1, _T), F32),
    )(aggp, h, W_upd_l, b_upd_l.reshape(1, _H), W_lin1, b_lin1.reshape(1, _H),
      W_lin2, b_lin2.reshape(1, _T))


def kernel(node_feats, edge_feats, edge_index, W_node, b_node, W_edge, b_edge,
           W_msg, b_msg, W_upd, b_upd, W_lin1, b_lin1, W_lin2, b_lin2):
    src2d = edge_index[0].reshape(_E // _K, _K)
    dst2d = edge_index[1].reshape(_E // _K, _K)
    C_all = jnp.einsum('ij,ljk->lik', W_edge, W_msg)
    d_all = jnp.einsum('j,ljk->lk', b_edge, W_msg) + b_msg
    C_lo = jnp.zeros((_L, 128, 256), F32)
    C_hi = jnp.zeros((_L, 128, 256), F32)
    for e8 in range(8):
        rs, cs = e8 * _ED, e8 * 32
        C_lo = C_lo.at[:, rs:rs + _ED, cs:cs + 32].set(C_all[:, :, :32])
        C_hi = C_hi.at[:, rs:rs + _ED, cs:cs + 32].set(C_all[:, :, 32:])
    d_lo = jnp.tile(d_all[:, :32], (1, 8))
    d_hi = jnp.tile(d_all[:, 32:], (1, 8))
    ef8 = edge_feats.reshape(_E8, 8 * _ED)
    zeros_n = jnp.zeros((_N, _H), F32)

    Q = _q_call(ef8, C_lo, C_hi, d_lo, d_hi)
    h, p = _embed_call(node_feats, W_node, b_node, W_msg[0])
    for l in range(_L):
        aggp = _sc_layers[l](p, Q, src2d, dst2d, zeros_n)
        if l < _L - 1:
            h, p = _upd_call(aggp, h, W_upd[l], b_upd[l], W_msg[l + 1])
        else:
            out = _fin_call(aggp, h, W_upd[l], b_upd[l],
                            W_lin1, b_lin1, W_lin2, b_lin2)
    return out


# speedup vs baseline: 10.1164x; 1.1838x over previous
import functools

import jax
import jax.numpy as jnp
from jax import lax
from jax.experimental import pallas as pl
from jax.experimental.pallas import tpu as pltpu
from jax.experimental.pallas import tpu_sc as plsc

F32 = jnp.float32

_N = 10000
_E = 320000
_ND = 128
_ED = 16
_H = 64
_L = 4
_T = 1

_NC = 2
_NS = 16
_NW = _NC * _NS
_EPW = _E // _NW
_K = 80
_NCHUNK = _EPW // _K
_RPT = _N // _NS


def _leaky(x):
    return jnp.maximum(x, 0.01 * x)


_E4 = _E // 4
_BE4 = 8000


def _pack2(lo, hi):
    ulo = jax.lax.bitcast_convert_type(
        lo.astype(jnp.bfloat16), jnp.uint16).astype(jnp.uint32)
    uhi = jax.lax.bitcast_convert_type(
        hi.astype(jnp.bfloat16), jnp.uint16).astype(jnp.uint32)
    return ulo | (uhi << 16)


def _q_body(ef_ref, clo_ref, chi_ref, dlo_ref, dhi_ref, q_ref):
    x = ef_ref[...]
    for l in range(_L):
        lo = jnp.dot(x, clo_ref[l], preferred_element_type=F32) + dlo_ref[l]
        hi = jnp.dot(x, chi_ref[l], preferred_element_type=F32) + dhi_ref[l]
        q_ref[l] = _pack2(lo, hi)


def _q_call(ef4, C_lo, C_hi, d_lo, d_hi):
    return pl.pallas_call(
        _q_body,
        grid=(_E4 // _BE4,),
        in_specs=[
            pl.BlockSpec((_BE4, 64), lambda i: (i, 0)),
            pl.BlockSpec((_L, 64, 128), lambda i: (0, 0, 0)),
            pl.BlockSpec((_L, 64, 128), lambda i: (0, 0, 0)),
            pl.BlockSpec((_L, 128), lambda i: (0, 0)),
            pl.BlockSpec((_L, 128), lambda i: (0, 0)),
        ],
        out_specs=pl.BlockSpec((_L, _BE4, 128), lambda i: (0, i, 0)),
        out_shape=jax.ShapeDtypeStruct((_L, _E4, 128), jnp.uint32),
    )(ef4, C_lo, C_hi, d_lo, d_hi)


def _embed_body(nf_ref, wn_ref, bn_ref, wm0_ref, h_ref, p_ref):
    h = jnp.dot(nf_ref[...], wn_ref[...], preferred_element_type=F32) + bn_ref[...]
    h_ref[...] = h
    wm = wm0_ref[...]
    p_ref[...] = _pack2(jnp.dot(h, wm[:, :32], preferred_element_type=F32),
                        jnp.dot(h, wm[:, 32:], preferred_element_type=F32))


def _embed_call(node_feats, W_node, b_node, W_msg0):
    return pl.pallas_call(
        _embed_body,
        out_shape=[
            jax.ShapeDtypeStruct((_N, _H), F32),
            jax.ShapeDtypeStruct((_N, _H // 2), jnp.uint32),
        ],
    )(node_feats, W_node, b_node.reshape(1, _H), W_msg0)


_R = 5
_TPC = _NCHUNK // _R
_LAG = 2


_KH = _K // 4


def _sc_body(l, p_hbm, q_hbm, s_hbm, d_hbm, z_hbm, out_hbm,
             agg_tab, sidx, didx, qbuf, pbuf, mbuf, *sems):
    qsem = sems[0:_R]
    psem = sems[_R:2 * _R]
    ssem = sems[2 * _R:3 * _R]
    c = lax.axis_index("c")
    s = lax.axis_index("s")
    row0 = s * _RPT
    pltpu.sync_copy(z_hbm.at[pl.ds(row0, _RPT)], agg_tab.at[pl.ds(row0, _RPT)])
    rowbase = (c * _NS + s) * _NCHUNK
    pltpu.sync_copy(s_hbm.at[pl.ds(rowbase, _NCHUNK)], sidx)
    pltpu.sync_copy(d_hbm.at[pl.ds(rowbase, _NCHUNK)], didx)
    plsc.subcore_barrier()

    def issue_loads(k, b):
        pltpu.async_copy(q_hbm.at[l, pl.ds((rowbase + k) * _KH, _KH)],
                         qbuf.at[b], qsem[b])
        pltpu.async_copy(p_hbm.at[sidx.at[k]], pbuf.at[b], psem[b])

    def wait_loads(k, b):
        pltpu.make_async_copy(q_hbm.at[l, pl.ds((rowbase + k) * _KH, _KH)],
                              qbuf.at[b], qsem[b]).wait()
        pltpu.make_async_copy(p_hbm.at[sidx.at[k]], pbuf.at[b], psem[b]).wait()

    def issue_scatter(k, b):
        pltpu.async_copy(mbuf.at[b], agg_tab.at[didx.at[k]], ssem[b], add=True)

    def wait_scatter(k, b):
        pltpu.make_async_copy(mbuf.at[b], agg_tab.at[didx.at[k]],
                              ssem[b]).wait()

    def compute(b):
        @plsc.parallel_loop(0, _K, 1, unroll=4)
        def _pl_body(r):
            for half in range(2):
                qw = qbuf[b, r // 4, pl.ds((r % 4) * 32 + half * 16, 16)]
                pw = pbuf[b, r, pl.ds(half * 16, 16)]
                qa, qb_ = plsc.unpack(plsc.bitcast(qw, jnp.bfloat16),
                                      format=plsc.PackFormat.INTERLEAVED)
                pa, pb_ = plsc.unpack(plsc.bitcast(pw, jnp.bfloat16),
                                      format=plsc.PackFormat.INTERLEAVED)
                x0 = qa + pa
                x1 = qb_ + pb_
                mbuf[b, r, pl.ds(half * 16, 16)] = jnp.maximum(x0, 0.01 * x0)
                mbuf[b, r, pl.ds(half * 16 + 32, 16)] = (
                    jnp.maximum(x1, 0.01 * x1))

    def tail(k, b):
        j = k - _LAG
        bj = (b - _LAG) % _R
        wait_scatter(j, bj)
        issue_loads(j + _R, bj)

    def step(k, b, do_tail):
        wait_loads(k, b)
        compute(b)
        issue_scatter(k, b)
        if do_tail:
            tail(k, b)

    for b in range(_R):
        issue_loads(b, b)
    for b in range(_R):
        step(b, b, do_tail=(b >= _LAG))

    def outer(t, _):
        for b in range(_R):
            step(t * _R + b, b, do_tail=True)
        return 0

    lax.fori_loop(1, _TPC - 1, outer, 0)

    for b in range(_R):
        k = (_TPC - 1) * _R + b
        step(k, b, do_tail=(k - _LAG + _R < _NCHUNK))
        if not (k - _LAG + _R < _NCHUNK):
            wait_scatter(k - _LAG, (b - _LAG) % _R)
    for b in range(_R - _LAG, _R):
        wait_scatter((_TPC - 1) * _R + b, b)

    plsc.subcore_barrier()
    pltpu.sync_copy(agg_tab.at[pl.ds(row0, _RPT)],
                    out_hbm.at[pl.ds(c * _N + row0, _RPT)])


_sc_layers = [
    functools.partial(
        pl.kernel,
        out_type=jax.ShapeDtypeStruct((_NC * _N, _H), F32),
        mesh=plsc.VectorSubcoreMesh(core_axis_name="c", subcore_axis_name="s"),
        compiler_params=pltpu.CompilerParams(use_tc_tiling_on_sc=False,
                                             needs_layout_passes=False),
        scratch_types=[
            pltpu.VMEM_SHARED((_N, _H), F32),
            pltpu.VMEM((_NCHUNK, _K), jnp.int32),
            pltpu.VMEM((_NCHUNK, _K), jnp.int32),
            pltpu.VMEM((_R, _KH, 128), jnp.uint32),
            pltpu.VMEM((_R, _K, _H // 2), jnp.uint32),
            pltpu.VMEM((_R, _K, _H), F32),
        ] + [pltpu.SemaphoreType.DMA] * (3 * _R),
    )(functools.partial(_sc_body, _lyr))
    for _lyr in range(_L)
]


def _upd_body(aggp_ref, h_ref, wu_ref, bu_ref, wm_ref, hn_ref, pn_ref):
    a = aggp_ref[...]
    agg = a[:_N] + a[_N:]
    t = jnp.dot(agg, wu_ref[...], preferred_element_type=F32) + bu_ref[...]
    hn = h_ref[...] + _leaky(t)
    hn_ref[...] = hn
    wm = wm_ref[...]
    pn_ref[...] = _pack2(jnp.dot(hn, wm[:, :32], preferred_element_type=F32),
                         jnp.dot(hn, wm[:, 32:], preferred_element_type=F32))


def _upd_call(aggp, h, W_upd_l, b_upd_l, W_msg_next):
    return pl.pallas_call(
        _upd_body,
        out_shape=[
            jax.ShapeDtypeStruct((_N, _H), F32),
            jax.ShapeDtypeStruct((_N, _H // 2), jnp.uint32),
        ],
    )(aggp, h, W_upd_l, b_upd_l.reshape(1, _H), W_msg_next)


def _fin_body(aggp_ref, h_ref, wu_ref, bu_ref, w1_ref, b1_ref, w2_ref, b2_ref,
              o_ref):
    a = aggp_ref[...]
    agg = a[:_N] + a[_N:]
    t = jnp.dot(agg, wu_ref[...], preferred_element_type=F32) + bu_ref[...]
    hn = h_ref[...] + _leaky(t)
    g = jnp.sum(hn, axis=0, keepdims=True)
    g = _leaky(jnp.dot(g, w1_ref[...], preferred_element_type=F32) + b1_ref[...])
    o_ref[...] = jnp.dot(g, w2_ref[...], preferred_element_type=F32) + b2_ref[...]


def _fin_call(aggp, h, W_upd_l, b_upd_l, W_lin1, b_lin1, W_lin2, b_lin2):
    return pl.pallas_call(
        _fin_body,
        out_shape=jax.ShapeDtypeStruct((1, _T), F32),
    )(aggp, h, W_upd_l, b_upd_l.reshape(1, _H), W_lin1, b_lin1.reshape(1, _H),
      W_lin2, b_lin2.reshape(1, _T))


def kernel(node_feats, edge_feats, edge_index, W_node, b_node, W_edge, b_edge,
           W_msg, b_msg, W_upd, b_upd, W_lin1, b_lin1, W_lin2, b_lin2):
    src2d = edge_index[0].reshape(_E // _K, _K)
    dst2d = edge_index[1].reshape(_E // _K, _K)
    C_all = jnp.einsum('ij,ljk->lik', W_edge, W_msg)
    d_all = jnp.einsum('j,ljk->lk', b_edge, W_msg) + b_msg
    eye4 = jnp.eye(4, dtype=F32)
    C_lo = jnp.einsum('ab,lic->labic', eye4,
                      C_all[:, :, :32]).reshape(_L, 64, 128)
    C_hi = jnp.einsum('ab,lic->labic', eye4,
                      C_all[:, :, 32:]).reshape(_L, 64, 128)
    d_lo = jnp.tile(d_all[:, :32], (1, 4))
    d_hi = jnp.tile(d_all[:, 32:], (1, 4))
    ef4 = edge_feats.reshape(_E4, 4 * _ED)
    zeros_n = jnp.zeros((_N, _H), F32)

    Q = _q_call(ef4, C_lo, C_hi, d_lo, d_hi)
    h, p = _embed_call(node_feats, W_node, b_node, W_msg[0])
    for l in range(_L):
        aggp = _sc_layers[l](p, Q, src2d, dst2d, zeros_n)
        if l < _L - 1:
            h, p = _upd_call(aggp, h, W_upd[l], b_upd[l], W_msg[l + 1])
        else:
            out = _fin_call(aggp, h, W_upd[l], b_upd[l],
                            W_lin1, b_lin1, W_lin2, b_lin2)
    return out
